# baseline (speedup 1.0000x reference)
import functools

import jax
import jax.numpy as jnp
from jax import lax
from jax.experimental import pallas as pl
from jax.experimental.pallas import tpu as pltpu
from jax.experimental.pallas import tpu_sc as plsc

_NC = 2
_NS = 16
_NW = _NC * _NS
_GRAN = 128 * _NW
_BUF_BYTES = 393216
_NEG_INV_2KS2 = -12.5

_N0, _N1, _N2 = 10000, 2560, 640


def _sc_gather(table, idx):
    _, D = table.shape
    B = idx.shape[0]
    bpw = B // _NW
    nch = 1
    while (bpw // nch) * D * 4 > _BUF_BYTES:
        nch *= 2
    chunk = bpw // nch
    idx3 = idx.reshape(_NW, nch, chunk)
    mesh = plsc.VectorSubcoreMesh(core_axis_name="c", subcore_axis_name="s")

    @functools.partial(
        pl.kernel,
        mesh=mesh,
        out_type=jax.ShapeDtypeStruct((B, D), jnp.float32),
        scratch_types=[
            pltpu.VMEM((nch, chunk), jnp.int32),
            pltpu.VMEM((chunk, D), jnp.float32),
            pltpu.SemaphoreType.DMA,
        ],
        compiler_params=pltpu.CompilerParams(use_tc_tiling_on_sc=False),
    )
    def gather_kernel(table_hbm, idx_hbm, out_hbm, idx_v, rows_v, sem):
        wid = lax.axis_index("s") * _NC + lax.axis_index("c")
        pltpu.sync_copy(idx_hbm.at[wid], idx_v)
        base = wid * bpw
        for j in range(nch):
            pltpu.async_copy(table_hbm.at[idx_v.at[j]], rows_v, sem).wait()
            pltpu.sync_copy(rows_v, out_hbm.at[pl.ds(base + j * chunk, chunk)])

    return gather_kernel(table, idx3)


def _gather_table(table, idx2d):
    n, k = idx2d.shape
    b = n * k
    idx = idx2d.T.reshape(-1).astype(jnp.int32)
    bp = -(-b // _GRAN) * _GRAN
    if bp > b:
        idx = jnp.concatenate([idx, jnp.zeros((bp - b,), jnp.int32)])
    rows = _sc_gather(table, idx)
    return rows[:b].reshape(k, n, table.shape[1])


def _pack_table(feats, coords):
    n = feats.shape[0]
    d = feats.shape[1] + 3
    dp = -(-d // 16) * 16
    return jnp.concatenate(
        [feats, coords, jnp.zeros((n, dp - d), jnp.float32)], axis=1
    )


def _pad_rows(a, n):
    return jnp.concatenate(
        [a, jnp.zeros((n - a.shape[0],) + a.shape[1:], a.dtype)]
    ) if a.shape[0] < n else a


def _bf(a):
    return a.astype(jnp.bfloat16).astype(jnp.float32)


def _dot(a, b):
    return jnp.dot(a.astype(jnp.bfloat16), b.astype(jnp.bfloat16),
                   preferred_element_type=jnp.float32)


def _gib_body(rows_ref, cdst_ref, obs_ref, w_ref, b_ref, out_ref, *, k, c, bn):
    cd = cdst_ref[...]
    o0 = _bf(obs_ref[0:1, :])
    o1 = _bf(obs_ref[1:2, :])
    o2 = _bf(obs_ref[2:3, :])

    def body(j, accs):
        rj = rows_ref[pl.ds(j, 1)][0]
        rel = _bf(rj[:, c:c + 3] - cd)
        dj = rel[:, 0:1] * o0 + rel[:, 1:2] * o1 + rel[:, 2:3] * o2
        wj = _bf(jnp.exp(dj * dj * _NEG_INV_2KS2))
        fj = _bf(rj[:, 0:c])
        return tuple(a + wj[:, o:o + 1] * fj for o, a in enumerate(accs))

    init = tuple(jnp.zeros((bn, c), jnp.float32) for _ in range(16))
    accs = lax.fori_loop(0, k, body, init)
    agg = jnp.concatenate(accs, axis=1) * (1.0 / k)
    out = _dot(agg, w_ref[...])
    out_ref[...] = jnp.maximum(out + b_ref[...], 0.0)


def _gib_call(rows3, cdst, obs8, w, b, bn):
    k, n, dp = rows3.shape
    c = w.shape[0] // 16
    m = w.shape[1]
    grid = n // bn
    body = functools.partial(_gib_body, k=k, c=c, bn=bn)
    return pl.pallas_call(
        body,
        grid=(grid,),
        in_specs=[
            pl.BlockSpec((k, bn, dp), lambda i: (0, i, 0)),
            pl.BlockSpec((bn, 3), lambda i: (i, 0)),
            pl.BlockSpec((8, 16), lambda i: (0, 0)),
            pl.BlockSpec((16 * c, m), lambda i: (0, 0)),
            pl.BlockSpec((1, m), lambda i: (0, 0)),
        ],
        out_specs=pl.BlockSpec((bn, m), lambda i: (i, 0)),
        out_shape=jax.ShapeDtypeStruct((n, m), jnp.float32),
    )(rows3, cdst, obs8, w, b.reshape(1, m))


def _unpool_body(rows_ref, clow_ref, skip_ref, wa_ref, wb_ref, b_ref, out_ref,
                 *, c):
    cl = clow_ref[...]
    ws = []
    for j in range(3):
        df = rows_ref[j][:, c:c + 3] - cl
        dd = jnp.sqrt(
            df[:, 0:1] * df[:, 0:1]
            + df[:, 1:2] * df[:, 1:2]
            + df[:, 2:3] * df[:, 2:3]
            + 1e-12
        )
        ws.append(1.0 / (dd + 1e-8))
    wsum = ws[0] + ws[1] + ws[2]
    interp = sum((ws[j] / wsum) * rows_ref[j][:, 0:c] for j in range(3))
    out = _dot(interp, wa_ref[...]) + _dot(skip_ref[...], wb_ref[...]) + b_ref[...]
    out_ref[...] = jnp.maximum(out, 0.0)


def _unpool_call(rows3, clow, skip, w, b, bn):
    _, n, dp = rows3.shape
    cs = skip.shape[1]
    c = w.shape[0] - cs
    m = w.shape[1]
    body = functools.partial(_unpool_body, c=c)
    return pl.pallas_call(
        body,
        grid=(n // bn,),
        in_specs=[
            pl.BlockSpec((3, bn, dp), lambda i: (0, i, 0)),
            pl.BlockSpec((bn, 3), lambda i: (i, 0)),
            pl.BlockSpec((bn, cs), lambda i: (i, 0)),
            pl.BlockSpec((c, m), lambda i: (0, 0)),
            pl.BlockSpec((cs, m), lambda i: (0, 0)),
            pl.BlockSpec((1, m), lambda i: (0, 0)),
        ],
        out_specs=pl.BlockSpec((bn, m), lambda i: (i, 0)),
        out_shape=jax.ShapeDtypeStruct((n, m), jnp.float32),
    )(rows3, clow, skip, w[:c], w[c:], b.reshape(1, m))


def _head_stats_body(d0_ref, wh1_ref, bh1_ref, h_ref, st_ref):
    h = _dot(d0_ref[...], wh1_ref[...]) + bh1_ref[...]
    h_ref[...] = h
    s = jnp.concatenate(
        [
            jnp.sum(h, axis=0, keepdims=True),
            jnp.sum(h * h, axis=0, keepdims=True),
            jnp.zeros((6, h.shape[1]), jnp.float32),
        ]
    )

    @pl.when(pl.program_id(0) == 0)
    def _():
        st_ref[...] = s

    @pl.when(pl.program_id(0) != 0)
    def _():
        st_ref[...] = st_ref[...] + s


def _head_out_body(h_ref, st_ref, g_ref, bb_ref, wh2_ref, bh2_ref, out_ref, *, n):
    mu = st_ref[0:1, :] * (1.0 / n)
    var = st_ref[1:2, :] * (1.0 / n) - mu * mu
    h = (h_ref[...] - mu) / jnp.sqrt(var + 1e-5) * g_ref[...] + bb_ref[...]
    h = jnp.maximum(h, 0.0)
    out_ref[...] = _dot(h, wh2_ref[...]) + bh2_ref[...]


def _head_call(d0, wh1, bh1, g_bn, b_bn, wh2, bh2, bn):
    n, m = d0.shape
    mo = wh2.shape[1]
    grid = n // bn
    h, st = pl.pallas_call(
        _head_stats_body,
        grid=(grid,),
        in_specs=[
            pl.BlockSpec((bn, m), lambda i: (i, 0)),
            pl.BlockSpec((m, m), lambda i: (0, 0)),
            pl.BlockSpec((1, m), lambda i: (0, 0)),
        ],
        out_specs=[
            pl.BlockSpec((bn, m), lambda i: (i, 0)),
            pl.BlockSpec((8, m), lambda i: (0, 0)),
        ],
        out_shape=[
            jax.ShapeDtypeStruct((n, m), jnp.float32),
            jax.ShapeDtypeStruct((8, m), jnp.float32),
        ],
    )(d0, wh1, bh1.reshape(1, m))
    body = functools.partial(_head_out_body, n=n)
    return pl.pallas_call(
        body,
        grid=(grid,),
        in_specs=[
            pl.BlockSpec((bn, m), lambda i: (i, 0)),
            pl.BlockSpec((8, m), lambda i: (0, 0)),
            pl.BlockSpec((1, m), lambda i: (0, 0)),
            pl.BlockSpec((1, m), lambda i: (0, 0)),
            pl.BlockSpec((m, mo), lambda i: (0, 0)),
            pl.BlockSpec((1, mo), lambda i: (0, 0)),
        ],
        out_specs=pl.BlockSpec((bn, mo), lambda i: (i, 0)),
        out_shape=jax.ShapeDtypeStruct((n, mo), jnp.float32),
    )(h, st, g_bn.reshape(1, m), b_bn.reshape(1, m), wh2, bh2.reshape(1, mo))


def kernel(x, neigh0, neigh1, neigh2, sub0, sub1, up0, up1, obs,
           W0, b0, Wp0, bp0, W1, b1, Wp1, bp1, W2, b2,
           Wd1, bd1, Wd0, bd0, Wh1, bh1, g_bn, b_bn, Wh2, bh2):
    coords0 = x[:, :3]
    feats = x[:, 3:]
    obs8 = jnp.zeros((8, 16), jnp.float32).at[:3].set(obs)

    rows = _gather_table(_pack_table(feats, coords0), neigh0)
    f0 = _gib_call(rows, coords0, obs8, W0, b0, 400)

    rows = _gather_table(_pack_table(f0, coords0), _pad_rows(sub0, _N1))
    coords1 = rows[0, :, 64:67]
    p1 = _gib_call(rows, coords1, obs8, Wp0, bp0, 320)

    rows = _gather_table(_pack_table(p1, coords1), _pad_rows(neigh1, _N1))
    f1 = _gib_call(rows, coords1, obs8, W1, b1, 320)

    rows = _gather_table(_pack_table(f1, coords1), _pad_rows(sub1, _N2))
    coords2 = rows[0, :, 128:131]
    p2 = _gib_call(rows, coords2, obs8, Wp1, bp1, 320)

    rows = _gather_table(_pack_table(p2, coords2), _pad_rows(neigh2, _N2))
    f2 = _gib_call(rows, coords2, obs8, W2, b2, 320)

    rows = _gather_table(_pack_table(f2, coords2), _pad_rows(up1, _N1))
    d1 = _unpool_call(rows, coords1, f1, Wd1, bd1, 320)

    rows = _gather_table(_pack_table(d1, coords1), up0)
    d0 = _unpool_call(rows, coords0, f0, Wd0, bd0, 400)

    return _head_call(d0, Wh1, bh1, g_bn, b_bn, Wh2, bh2, 400)


# speedup vs baseline: 1.0015x; 1.0015x over previous
import functools

import jax
import jax.numpy as jnp
from jax import lax
from jax.experimental import pallas as pl
from jax.experimental.pallas import tpu as pltpu
from jax.experimental.pallas import tpu_sc as plsc

_NC = 2
_NS = 16
_NW = _NC * _NS
_GRAN = 128 * _NW
_BUF_BYTES = 196608
_NEG_INV_2KS2 = -12.5

_N0, _N1, _N2 = 10000, 2560, 640


def _sc_gather(table, idx):
    _, D = table.shape
    B = idx.shape[0]
    bpw = B // _NW
    nch = 2
    while (bpw // nch) * D * 4 > _BUF_BYTES:
        nch *= 2
    chunk = bpw // nch
    idx3 = idx.reshape(_NW, nch, chunk)
    mesh = plsc.VectorSubcoreMesh(core_axis_name="c", subcore_axis_name="s")

    @functools.partial(
        pl.kernel,
        mesh=mesh,
        out_type=jax.ShapeDtypeStruct((B, D), jnp.float32),
        scratch_types=[
            pltpu.VMEM((nch, chunk), jnp.int32),
            pltpu.VMEM((chunk, D), jnp.float32),
            pltpu.VMEM((chunk, D), jnp.float32),
            pltpu.SemaphoreType.DMA,
            pltpu.SemaphoreType.DMA,
        ],
        compiler_params=pltpu.CompilerParams(use_tc_tiling_on_sc=False),
    )
    def gather_kernel(table_hbm, idx_hbm, out_hbm, idx_v, rows0, rows1, sem0, sem1):
        wid = lax.axis_index("s") * _NC + lax.axis_index("c")
        pltpu.sync_copy(idx_hbm.at[wid], idx_v)
        base = wid * bpw
        bufs = (rows0, rows1)
        sems = (sem0, sem1)
        copies = []
        for j in range(nch):
            copies.append(
                pltpu.async_copy(table_hbm.at[idx_v.at[j]], bufs[j % 2], sems[j % 2])
            )
            if j >= 1:
                copies[j - 1].wait()
                pltpu.sync_copy(
                    bufs[(j - 1) % 2],
                    out_hbm.at[pl.ds(base + (j - 1) * chunk, chunk)],
                )
        copies[nch - 1].wait()
        pltpu.sync_copy(
            bufs[(nch - 1) % 2],
            out_hbm.at[pl.ds(base + (nch - 1) * chunk, chunk)],
        )

    return gather_kernel(table, idx3)


def _gather_table(table, idx2d):
    n, k = idx2d.shape
    b = n * k
    idx = idx2d.T.reshape(-1).astype(jnp.int32)
    bp = -(-b // _GRAN) * _GRAN
    if bp > b:
        idx = jnp.concatenate([idx, jnp.zeros((bp - b,), jnp.int32)])
    rows = _sc_gather(table, idx)
    return rows[:b].reshape(k, n, table.shape[1])


def _pack_table(feats, coords):
    n = feats.shape[0]
    d = feats.shape[1] + 3
    dp = -(-d // 16) * 16
    return jnp.concatenate(
        [feats, coords, jnp.zeros((n, dp - d), jnp.float32)], axis=1
    )


def _pad_rows(a, n):
    return jnp.concatenate(
        [a, jnp.zeros((n - a.shape[0],) + a.shape[1:], a.dtype)]
    ) if a.shape[0] < n else a


def _bf(a):
    return a.astype(jnp.bfloat16).astype(jnp.float32)


def _dot(a, b):
    return jnp.dot(a.astype(jnp.bfloat16), b.astype(jnp.bfloat16),
                   preferred_element_type=jnp.float32)


def _gib_body(rows_ref, cdst_ref, obs_ref, w_ref, b_ref, out_ref, *, k, c, bn):
    cd = cdst_ref[...]
    o0 = _bf(obs_ref[0:1, :])
    o1 = _bf(obs_ref[1:2, :])
    o2 = _bf(obs_ref[2:3, :])

    def body(j, accs):
        rj = rows_ref[pl.ds(j, 1)][0]
        rel = _bf(rj[:, c:c + 3] - cd)
        dj = rel[:, 0:1] * o0 + rel[:, 1:2] * o1 + rel[:, 2:3] * o2
        wj = _bf(jnp.exp(dj * dj * _NEG_INV_2KS2))
        fj = _bf(rj[:, 0:c])
        return tuple(a + wj[:, o:o + 1] * fj for o, a in enumerate(accs))

    init = tuple(jnp.zeros((bn, c), jnp.float32) for _ in range(16))
    accs = lax.fori_loop(0, k, body, init)
    agg = jnp.concatenate(accs, axis=1) * (1.0 / k)
    out = _dot(agg, w_ref[...])
    out_ref[...] = jnp.maximum(out + b_ref[...], 0.0)


def _gib_call(rows3, cdst, obs8, w, b, bn):
    k, n, dp = rows3.shape
    c = w.shape[0] // 16
    m = w.shape[1]
    grid = n // bn
    body = functools.partial(_gib_body, k=k, c=c, bn=bn)
    return pl.pallas_call(
        body,
        grid=(grid,),
        in_specs=[
            pl.BlockSpec((k, bn, dp), lambda i: (0, i, 0)),
            pl.BlockSpec((bn, 3), lambda i: (i, 0)),
            pl.BlockSpec((8, 16), lambda i: (0, 0)),
            pl.BlockSpec((16 * c, m), lambda i: (0, 0)),
            pl.BlockSpec((1, m), lambda i: (0, 0)),
        ],
        out_specs=pl.BlockSpec((bn, m), lambda i: (i, 0)),
        out_shape=jax.ShapeDtypeStruct((n, m), jnp.float32),
    )(rows3, cdst, obs8, w, b.reshape(1, m))


def _unpool_body(rows_ref, clow_ref, skip_ref, wa_ref, wb_ref, b_ref, out_ref,
                 *, c):
    cl = clow_ref[...]
    ws = []
    for j in range(3):
        df = rows_ref[j][:, c:c + 3] - cl
        dd = jnp.sqrt(
            df[:, 0:1] * df[:, 0:1]
            + df[:, 1:2] * df[:, 1:2]
            + df[:, 2:3] * df[:, 2:3]
            + 1e-12
        )
        ws.append(1.0 / (dd + 1e-8))
    wsum = ws[0] + ws[1] + ws[2]
    interp = sum((ws[j] / wsum) * rows_ref[j][:, 0:c] for j in range(3))
    out = _dot(interp, wa_ref[...]) + _dot(skip_ref[...], wb_ref[...]) + b_ref[...]
    out_ref[...] = jnp.maximum(out, 0.0)


def _unpool_call(rows3, clow, skip, w, b, bn):
    _, n, dp = rows3.shape
    cs = skip.shape[1]
    c = w.shape[0] - cs
    m = w.shape[1]
    body = functools.partial(_unpool_body, c=c)
    return pl.pallas_call(
        body,
        grid=(n // bn,),
        in_specs=[
            pl.BlockSpec((3, bn, dp), lambda i: (0, i, 0)),
            pl.BlockSpec((bn, 3), lambda i: (i, 0)),
            pl.BlockSpec((bn, cs), lambda i: (i, 0)),
            pl.BlockSpec((c, m), lambda i: (0, 0)),
            pl.BlockSpec((cs, m), lambda i: (0, 0)),
            pl.BlockSpec((1, m), lambda i: (0, 0)),
        ],
        out_specs=pl.BlockSpec((bn, m), lambda i: (i, 0)),
        out_shape=jax.ShapeDtypeStruct((n, m), jnp.float32),
    )(rows3, clow, skip, w[:c], w[c:], b.reshape(1, m))


def _head_stats_body(d0_ref, wh1_ref, bh1_ref, h_ref, st_ref):
    h = _dot(d0_ref[...], wh1_ref[...]) + bh1_ref[...]
    h_ref[...] = h
    s = jnp.concatenate(
        [
            jnp.sum(h, axis=0, keepdims=True),
            jnp.sum(h * h, axis=0, keepdims=True),
            jnp.zeros((6, h.shape[1]), jnp.float32),
        ]
    )

    @pl.when(pl.program_id(0) == 0)
    def _():
        st_ref[...] = s

    @pl.when(pl.program_id(0) != 0)
    def _():
        st_ref[...] = st_ref[...] + s


def _head_out_body(h_ref, st_ref, g_ref, bb_ref, wh2_ref, bh2_ref, out_ref, *, n):
    mu = st_ref[0:1, :] * (1.0 / n)
    var = st_ref[1:2, :] * (1.0 / n) - mu * mu
    h = (h_ref[...] - mu) / jnp.sqrt(var + 1e-5) * g_ref[...] + bb_ref[...]
    h = jnp.maximum(h, 0.0)
    out_ref[...] = _dot(h, wh2_ref[...]) + bh2_ref[...]


def _head_call(d0, wh1, bh1, g_bn, b_bn, wh2, bh2, bn):
    n, m = d0.shape
    mo = wh2.shape[1]
    grid = n // bn
    h, st = pl.pallas_call(
        _head_stats_body,
        grid=(grid,),
        in_specs=[
            pl.BlockSpec((bn, m), lambda i: (i, 0)),
            pl.BlockSpec((m, m), lambda i: (0, 0)),
            pl.BlockSpec((1, m), lambda i: (0, 0)),
        ],
        out_specs=[
            pl.BlockSpec((bn, m), lambda i: (i, 0)),
            pl.BlockSpec((8, m), lambda i: (0, 0)),
        ],
        out_shape=[
            jax.ShapeDtypeStruct((n, m), jnp.float32),
            jax.ShapeDtypeStruct((8, m), jnp.float32),
        ],
    )(d0, wh1, bh1.reshape(1, m))
    body = functools.partial(_head_out_body, n=n)
    return pl.pallas_call(
        body,
        grid=(grid,),
        in_specs=[
            pl.BlockSpec((bn, m), lambda i: (i, 0)),
            pl.BlockSpec((8, m), lambda i: (0, 0)),
            pl.BlockSpec((1, m), lambda i: (0, 0)),
            pl.BlockSpec((1, m), lambda i: (0, 0)),
            pl.BlockSpec((m, mo), lambda i: (0, 0)),
            pl.BlockSpec((1, mo), lambda i: (0, 0)),
        ],
        out_specs=pl.BlockSpec((bn, mo), lambda i: (i, 0)),
        out_shape=jax.ShapeDtypeStruct((n, mo), jnp.float32),
    )(h, st, g_bn.reshape(1, m), b_bn.reshape(1, m), wh2, bh2.reshape(1, mo))


def kernel(x, neigh0, neigh1, neigh2, sub0, sub1, up0, up1, obs,
           W0, b0, Wp0, bp0, W1, b1, Wp1, bp1, W2, b2,
           Wd1, bd1, Wd0, bd0, Wh1, bh1, g_bn, b_bn, Wh2, bh2):
    coords0 = x[:, :3]
    feats = x[:, 3:]
    obs8 = jnp.zeros((8, 16), jnp.float32).at[:3].set(obs)

    rows = _gather_table(_pack_table(feats, coords0), neigh0)
    f0 = _gib_call(rows, coords0, obs8, W0, b0, 400)

    rows = _gather_table(_pack_table(f0, coords0), _pad_rows(sub0, _N1))
    coords1 = rows[0, :, 64:67]
    p1 = _gib_call(rows, coords1, obs8, Wp0, bp0, 320)

    rows = _gather_table(_pack_table(p1, coords1), _pad_rows(neigh1, _N1))
    f1 = _gib_call(rows, coords1, obs8, W1, b1, 320)

    rows = _gather_table(_pack_table(f1, coords1), _pad_rows(sub1, _N2))
    coords2 = rows[0, :, 128:131]
    p2 = _gib_call(rows, coords2, obs8, Wp1, bp1, 320)

    rows = _gather_table(_pack_table(p2, coords2), _pad_rows(neigh2, _N2))
    f2 = _gib_call(rows, coords2, obs8, W2, b2, 320)

    rows = _gather_table(_pack_table(f2, coords2), _pad_rows(up1, _N1))
    d1 = _unpool_call(rows, coords1, f1, Wd1, bd1, 320)

    rows = _gather_table(_pack_table(d1, coords1), up0)
    d0 = _unpool_call(rows, coords0, f0, Wd0, bd0, 400)

    return _head_call(d0, Wh1, bh1, g_bn, b_bn, Wh2, bh2, 400)


# speedup vs baseline: 1.1870x; 1.1853x over previous
import functools

import jax
import jax.numpy as jnp
from jax import lax
from jax.experimental import pallas as pl
from jax.experimental.pallas import tpu as pltpu
from jax.experimental.pallas import tpu_sc as plsc

_NC = 2
_NS = 16
_NW = _NC * _NS
_GRAN = 128 * _NW
_BUF_BYTES = 196608
_NEG_INV_2KS2 = -12.5

_N0, _N1, _N2 = 10000, 2560, 640


def _sc_gather(table, idx):
    _, D = table.shape
    B = idx.shape[0]
    bpw = B // _NW
    nch = 2
    while (bpw // nch) * D * 4 > _BUF_BYTES:
        nch *= 2
    chunk = bpw // nch
    idx3 = idx.reshape(_NW, nch, chunk)
    mesh = plsc.VectorSubcoreMesh(core_axis_name="c", subcore_axis_name="s")

    @functools.partial(
        pl.kernel,
        mesh=mesh,
        out_type=jax.ShapeDtypeStruct((B, D), jnp.float32),
        scratch_types=[
            pltpu.VMEM((nch, chunk), jnp.int32),
            pltpu.VMEM((chunk, D), jnp.float32),
            pltpu.VMEM((chunk, D), jnp.float32),
            pltpu.SemaphoreType.DMA,
            pltpu.SemaphoreType.DMA,
        ],
        compiler_params=pltpu.CompilerParams(use_tc_tiling_on_sc=False),
    )
    def gather_kernel(table_hbm, idx_hbm, out_hbm, idx_v, rows0, rows1, sem0, sem1):
        wid = lax.axis_index("s") * _NC + lax.axis_index("c")
        pltpu.sync_copy(idx_hbm.at[wid], idx_v)
        base = wid * bpw
        bufs = (rows0, rows1)
        sems = (sem0, sem1)
        copies = []
        for j in range(nch):
            copies.append(
                pltpu.async_copy(table_hbm.at[idx_v.at[j]], bufs[j % 2], sems[j % 2])
            )
            if j >= 1:
                copies[j - 1].wait()
                pltpu.sync_copy(
                    bufs[(j - 1) % 2],
                    out_hbm.at[pl.ds(base + (j - 1) * chunk, chunk)],
                )
        copies[nch - 1].wait()
        pltpu.sync_copy(
            bufs[(nch - 1) % 2],
            out_hbm.at[pl.ds(base + (nch - 1) * chunk, chunk)],
        )

    return gather_kernel(table, idx3)


def _gather_table(table, idx2d):
    n, k = idx2d.shape
    b = n * k
    idx = idx2d.T.reshape(-1).astype(jnp.int32)
    bp = -(-b // _GRAN) * _GRAN
    if bp > b:
        idx = jnp.concatenate([idx, jnp.zeros((bp - b,), jnp.int32)])
    rows = _sc_gather(table, idx)
    return rows[:b].reshape(k, n, table.shape[1])


def _pack_table(feats, coords):
    n = feats.shape[0]
    d = feats.shape[1] + 3
    dp = -(-d // 16) * 16
    return jnp.concatenate(
        [feats, coords, jnp.zeros((n, dp - d), jnp.float32)], axis=1
    )


def _pad_rows(a, n):
    return jnp.concatenate(
        [a, jnp.zeros((n - a.shape[0],) + a.shape[1:], a.dtype)]
    ) if a.shape[0] < n else a


def _bf(a):
    return a.astype(jnp.bfloat16).astype(jnp.float32)


def _dot(a, b):
    return jnp.dot(a.astype(jnp.bfloat16), b.astype(jnp.bfloat16),
                   preferred_element_type=jnp.float32)


def _gib_body(rows_ref, cdst_ref, obs_ref, w_ref, b_ref, out_ref, *, k, c, bn):
    cd = cdst_ref[...]
    o0 = _bf(obs_ref[0:1, :])
    o1 = _bf(obs_ref[1:2, :])
    o2 = _bf(obs_ref[2:3, :])

    def one(j):
        rj = rows_ref[pl.ds(j, 1)][0]
        rel = _bf(rj[:, c:c + 3] - cd)
        dj = rel[:, 0:1] * o0 + rel[:, 1:2] * o1 + rel[:, 2:3] * o2
        wj = _bf(jnp.exp(dj * dj * _NEG_INV_2KS2))
        fj = _bf(rj[:, 0:c])
        return wj, fj

    if k % 2 == 0:
        def body(i, accs):
            wa, fa = one(2 * i)
            wb, fb = one(2 * i + 1)
            return tuple(
                a + wa[:, o:o + 1] * fa + wb[:, o:o + 1] * fb
                for o, a in enumerate(accs)
            )
        steps = k // 2
    else:
        def body(i, accs):
            wa, fa = one(i)
            return tuple(a + wa[:, o:o + 1] * fa for o, a in enumerate(accs))
        steps = k

    init = tuple(jnp.zeros((bn, c), jnp.float32) for _ in range(16))
    accs = lax.fori_loop(0, steps, body, init)
    agg = jnp.concatenate(accs, axis=1) * (1.0 / k)
    out = _dot(agg, w_ref[...])
    out_ref[...] = jnp.maximum(out + b_ref[...], 0.0)


def _gib_call(rows3, cdst, obs8, w, b, bn):
    k, n, dp = rows3.shape
    c = w.shape[0] // 16
    m = w.shape[1]
    grid = n // bn
    body = functools.partial(_gib_body, k=k, c=c, bn=bn)
    return pl.pallas_call(
        body,
        grid=(grid,),
        in_specs=[
            pl.BlockSpec((k, bn, dp), lambda i: (0, i, 0)),
            pl.BlockSpec((bn, 3), lambda i: (i, 0)),
            pl.BlockSpec((8, 16), lambda i: (0, 0)),
            pl.BlockSpec((16 * c, m), lambda i: (0, 0)),
            pl.BlockSpec((1, m), lambda i: (0, 0)),
        ],
        out_specs=pl.BlockSpec((bn, m), lambda i: (i, 0)),
        out_shape=jax.ShapeDtypeStruct((n, m), jnp.float32),
    )(rows3, cdst, obs8, w, b.reshape(1, m))


def _unpool_body(rows_ref, clow_ref, skip_ref, wa_ref, wb_ref, b_ref, out_ref,
                 *, c):
    cl = clow_ref[...]
    ws = []
    for j in range(3):
        df = rows_ref[j][:, c:c + 3] - cl
        dd = jnp.sqrt(
            df[:, 0:1] * df[:, 0:1]
            + df[:, 1:2] * df[:, 1:2]
            + df[:, 2:3] * df[:, 2:3]
            + 1e-12
        )
        ws.append(1.0 / (dd + 1e-8))
    wsum = ws[0] + ws[1] + ws[2]
    interp = sum((ws[j] / wsum) * rows_ref[j][:, 0:c] for j in range(3))
    out = _dot(interp, wa_ref[...]) + _dot(skip_ref[...], wb_ref[...]) + b_ref[...]
    out_ref[...] = jnp.maximum(out, 0.0)


def _unpool_call(rows3, clow, skip, w, b, bn):
    _, n, dp = rows3.shape
    cs = skip.shape[1]
    c = w.shape[0] - cs
    m = w.shape[1]
    body = functools.partial(_unpool_body, c=c)
    return pl.pallas_call(
        body,
        grid=(n // bn,),
        in_specs=[
            pl.BlockSpec((3, bn, dp), lambda i: (0, i, 0)),
            pl.BlockSpec((bn, 3), lambda i: (i, 0)),
            pl.BlockSpec((bn, cs), lambda i: (i, 0)),
            pl.BlockSpec((c, m), lambda i: (0, 0)),
            pl.BlockSpec((cs, m), lambda i: (0, 0)),
            pl.BlockSpec((1, m), lambda i: (0, 0)),
        ],
        out_specs=pl.BlockSpec((bn, m), lambda i: (i, 0)),
        out_shape=jax.ShapeDtypeStruct((n, m), jnp.float32),
    )(rows3, clow, skip, w[:c], w[c:], b.reshape(1, m))


def _head_stats_body(d0_ref, wh1_ref, bh1_ref, h_ref, st_ref):
    h = _dot(d0_ref[...], wh1_ref[...]) + bh1_ref[...]
    h_ref[...] = h
    s = jnp.concatenate(
        [
            jnp.sum(h, axis=0, keepdims=True),
            jnp.sum(h * h, axis=0, keepdims=True),
            jnp.zeros((6, h.shape[1]), jnp.float32),
        ]
    )

    @pl.when(pl.program_id(0) == 0)
    def _():
        st_ref[...] = s

    @pl.when(pl.program_id(0) != 0)
    def _():
        st_ref[...] = st_ref[...] + s


def _head_out_body(h_ref, st_ref, g_ref, bb_ref, wh2_ref, bh2_ref, out_ref, *, n):
    mu = st_ref[0:1, :] * (1.0 / n)
    var = st_ref[1:2, :] * (1.0 / n) - mu * mu
    h = (h_ref[...] - mu) / jnp.sqrt(var + 1e-5) * g_ref[...] + bb_ref[...]
    h = jnp.maximum(h, 0.0)
    out_ref[...] = _dot(h, wh2_ref[...]) + bh2_ref[...]


def _head_call(d0, wh1, bh1, g_bn, b_bn, wh2, bh2, bn):
    n, m = d0.shape
    mo = wh2.shape[1]
    grid = n // bn
    h, st = pl.pallas_call(
        _head_stats_body,
        grid=(grid,),
        in_specs=[
            pl.BlockSpec((bn, m), lambda i: (i, 0)),
            pl.BlockSpec((m, m), lambda i: (0, 0)),
            pl.BlockSpec((1, m), lambda i: (0, 0)),
        ],
        out_specs=[
            pl.BlockSpec((bn, m), lambda i: (i, 0)),
            pl.BlockSpec((8, m), lambda i: (0, 0)),
        ],
        out_shape=[
            jax.ShapeDtypeStruct((n, m), jnp.float32),
            jax.ShapeDtypeStruct((8, m), jnp.float32),
        ],
    )(d0, wh1, bh1.reshape(1, m))
    body = functools.partial(_head_out_body, n=n)
    return pl.pallas_call(
        body,
        grid=(grid,),
        in_specs=[
            pl.BlockSpec((bn, m), lambda i: (i, 0)),
            pl.BlockSpec((8, m), lambda i: (0, 0)),
            pl.BlockSpec((1, m), lambda i: (0, 0)),
            pl.BlockSpec((1, m), lambda i: (0, 0)),
            pl.BlockSpec((m, mo), lambda i: (0, 0)),
            pl.BlockSpec((1, mo), lambda i: (0, 0)),
        ],
        out_specs=pl.BlockSpec((bn, mo), lambda i: (i, 0)),
        out_shape=jax.ShapeDtypeStruct((n, mo), jnp.float32),
    )(h, st, g_bn.reshape(1, m), b_bn.reshape(1, m), wh2, bh2.reshape(1, mo))


def kernel(x, neigh0, neigh1, neigh2, sub0, sub1, up0, up1, obs,
           W0, b0, Wp0, bp0, W1, b1, Wp1, bp1, W2, b2,
           Wd1, bd1, Wd0, bd0, Wh1, bh1, g_bn, b_bn, Wh2, bh2):
    coords0 = x[:, :3]
    feats = x[:, 3:]
    obs8 = jnp.zeros((8, 16), jnp.float32).at[:3].set(obs)

    rows = _gather_table(_pack_table(feats, coords0), neigh0)
    f0 = _gib_call(rows, coords0, obs8, W0, b0, 1000)

    rows = _gather_table(_pack_table(f0, coords0), _pad_rows(sub0, _N1))
    coords1 = rows[0, :, 64:67]
    p1 = _gib_call(rows, coords1, obs8, Wp0, bp0, 320)

    rows = _gather_table(_pack_table(p1, coords1), _pad_rows(neigh1, _N1))
    f1 = _gib_call(rows, coords1, obs8, W1, b1, 320)

    rows = _gather_table(_pack_table(f1, coords1), _pad_rows(sub1, _N2))
    coords2 = rows[0, :, 128:131]
    p2 = _gib_call(rows, coords2, obs8, Wp1, bp1, 320)

    rows = _gather_table(_pack_table(p2, coords2), _pad_rows(neigh2, _N2))
    f2 = _gib_call(rows, coords2, obs8, W2, b2, 320)

    rows = _gather_table(_pack_table(f2, coords2), _pad_rows(up1, _N1))
    d1 = _unpool_call(rows, coords1, f1, Wd1, bd1, 320)

    rows = _gather_table(_pack_table(d1, coords1), up0)
    d0 = _unpool_call(rows, coords0, f0, Wd0, bd0, 400)

    return _head_call(d0, Wh1, bh1, g_bn, b_bn, Wh2, bh2, 400)


# speedup vs baseline: 1.2526x; 1.0553x over previous
import functools

import jax
import jax.numpy as jnp
from jax import lax
from jax.experimental import pallas as pl
from jax.experimental.pallas import tpu as pltpu
from jax.experimental.pallas import tpu_sc as plsc

_NC = 2
_NS = 16
_NW = _NC * _NS
_GRAN = 128 * _NW
_BUF_BYTES = 196608
_NEG_INV_2KS2 = -12.5

_N0, _N1, _N2 = 10000, 2560, 640


def _sc_gather(table, idx):
    _, D = table.shape
    B = idx.shape[0]
    bpw = B // _NW
    nch = 2
    while (bpw // nch) * D * 4 > _BUF_BYTES:
        nch *= 2
    chunk = bpw // nch
    idx3 = idx.reshape(_NW, nch, chunk)
    mesh = plsc.VectorSubcoreMesh(core_axis_name="c", subcore_axis_name="s")

    @functools.partial(
        pl.kernel,
        mesh=mesh,
        out_type=jax.ShapeDtypeStruct((B, D), jnp.float32),
        scratch_types=[
            pltpu.VMEM((nch, chunk), jnp.int32),
            pltpu.VMEM((chunk, D), jnp.float32),
            pltpu.VMEM((chunk, D), jnp.float32),
            pltpu.SemaphoreType.DMA,
            pltpu.SemaphoreType.DMA,
        ],
        compiler_params=pltpu.CompilerParams(use_tc_tiling_on_sc=False),
    )
    def gather_kernel(table_hbm, idx_hbm, out_hbm, idx_v, rows0, rows1, sem0, sem1):
        wid = lax.axis_index("s") * _NC + lax.axis_index("c")
        pltpu.sync_copy(idx_hbm.at[wid], idx_v)
        base = wid * bpw
        bufs = (rows0, rows1)
        sems = (sem0, sem1)
        copies = []
        for j in range(nch):
            copies.append(
                pltpu.async_copy(table_hbm.at[idx_v.at[j]], bufs[j % 2], sems[j % 2])
            )
            if j >= 1:
                copies[j - 1].wait()
                pltpu.sync_copy(
                    bufs[(j - 1) % 2],
                    out_hbm.at[pl.ds(base + (j - 1) * chunk, chunk)],
                )
        copies[nch - 1].wait()
        pltpu.sync_copy(
            bufs[(nch - 1) % 2],
            out_hbm.at[pl.ds(base + (nch - 1) * chunk, chunk)],
        )

    return gather_kernel(table, idx3)


def _gather_table(table, idx2d):
    n, k = idx2d.shape
    b = n * k
    idx = idx2d.T.reshape(-1).astype(jnp.int32)
    bp = -(-b // _GRAN) * _GRAN
    if bp > b:
        idx = jnp.concatenate([idx, jnp.zeros((bp - b,), jnp.int32)])
    rows = _sc_gather(table, idx)
    return rows[:b].reshape(k, n, table.shape[1])


def _pack_table(feats, coords):
    n = feats.shape[0]
    d = feats.shape[1] + 3
    dp = -(-d // 16) * 16
    return jnp.concatenate(
        [feats, coords, jnp.zeros((n, dp - d), jnp.float32)], axis=1
    )


def _pad_rows(a, n):
    return jnp.concatenate(
        [a, jnp.zeros((n - a.shape[0],) + a.shape[1:], a.dtype)]
    ) if a.shape[0] < n else a


def _bf(a):
    return a.astype(jnp.bfloat16).astype(jnp.float32)


def _dot(a, b):
    return jnp.dot(a.astype(jnp.bfloat16), b.astype(jnp.bfloat16),
                   preferred_element_type=jnp.float32)


def _gib_body(rows_ref, cdst_ref, obs_ref, w_ref, b_ref, out_ref, *, k, c, bn):
    cd = cdst_ref[...]
    o0 = _bf(obs_ref[0:1, :])
    o1 = _bf(obs_ref[1:2, :])
    o2 = _bf(obs_ref[2:3, :])

    def one(j):
        rj = rows_ref[pl.ds(j, 1)][0]
        rel = _bf(rj[:, c:c + 3] - cd)
        dj = rel[:, 0:1] * o0 + rel[:, 1:2] * o1 + rel[:, 2:3] * o2
        wj = _bf(jnp.exp(dj * dj * _NEG_INV_2KS2))
        fj = _bf(rj[:, 0:c])
        return wj, fj

    unroll = 4 if k % 4 == 0 else (2 if k % 2 == 0 else 1)

    def body(i, accs):
        wf = [one(unroll * i + u) for u in range(unroll)]
        for w, f in wf:
            accs = tuple(a + w[:, o:o + 1] * f for o, a in enumerate(accs))
        return accs

    steps = k // unroll

    init = tuple(jnp.zeros((bn, c), jnp.float32) for _ in range(16))
    accs = lax.fori_loop(0, steps, body, init)
    agg = jnp.concatenate(accs, axis=1) * (1.0 / k)
    out = _dot(agg, w_ref[...])
    out_ref[...] = jnp.maximum(out + b_ref[...], 0.0)


def _gib_call(rows3, cdst, obs8, w, b, bn):
    k, n, dp = rows3.shape
    c = w.shape[0] // 16
    m = w.shape[1]
    grid = n // bn
    body = functools.partial(_gib_body, k=k, c=c, bn=bn)
    return pl.pallas_call(
        body,
        grid=(grid,),
        in_specs=[
            pl.BlockSpec((k, bn, dp), lambda i: (0, i, 0)),
            pl.BlockSpec((bn, 3), lambda i: (i, 0)),
            pl.BlockSpec((8, 16), lambda i: (0, 0)),
            pl.BlockSpec((16 * c, m), lambda i: (0, 0)),
            pl.BlockSpec((1, m), lambda i: (0, 0)),
        ],
        out_specs=pl.BlockSpec((bn, m), lambda i: (i, 0)),
        out_shape=jax.ShapeDtypeStruct((n, m), jnp.float32),
    )(rows3, cdst, obs8, w, b.reshape(1, m))


def _unpool_body(rows_ref, clow_ref, skip_ref, wa_ref, wb_ref, b_ref, out_ref,
                 *, c):
    cl = clow_ref[...]
    ws = []
    for j in range(3):
        df = rows_ref[j][:, c:c + 3] - cl
        dd = jnp.sqrt(
            df[:, 0:1] * df[:, 0:1]
            + df[:, 1:2] * df[:, 1:2]
            + df[:, 2:3] * df[:, 2:3]
            + 1e-12
        )
        ws.append(1.0 / (dd + 1e-8))
    wsum = ws[0] + ws[1] + ws[2]
    interp = sum((ws[j] / wsum) * rows_ref[j][:, 0:c] for j in range(3))
    out = _dot(interp, wa_ref[...]) + _dot(skip_ref[...], wb_ref[...]) + b_ref[...]
    out_ref[...] = jnp.maximum(out, 0.0)


def _unpool_call(rows3, clow, skip, w, b, bn):
    _, n, dp = rows3.shape
    cs = skip.shape[1]
    c = w.shape[0] - cs
    m = w.shape[1]
    body = functools.partial(_unpool_body, c=c)
    return pl.pallas_call(
        body,
        grid=(n // bn,),
        in_specs=[
            pl.BlockSpec((3, bn, dp), lambda i: (0, i, 0)),
            pl.BlockSpec((bn, 3), lambda i: (i, 0)),
            pl.BlockSpec((bn, cs), lambda i: (i, 0)),
            pl.BlockSpec((c, m), lambda i: (0, 0)),
            pl.BlockSpec((cs, m), lambda i: (0, 0)),
            pl.BlockSpec((1, m), lambda i: (0, 0)),
        ],
        out_specs=pl.BlockSpec((bn, m), lambda i: (i, 0)),
        out_shape=jax.ShapeDtypeStruct((n, m), jnp.float32),
    )(rows3, clow, skip, w[:c], w[c:], b.reshape(1, m))


def _head_stats_body(d0_ref, wh1_ref, bh1_ref, h_ref, st_ref):
    h = _dot(d0_ref[...], wh1_ref[...]) + bh1_ref[...]
    h_ref[...] = h
    s = jnp.concatenate(
        [
            jnp.sum(h, axis=0, keepdims=True),
            jnp.sum(h * h, axis=0, keepdims=True),
            jnp.zeros((6, h.shape[1]), jnp.float32),
        ]
    )

    @pl.when(pl.program_id(0) == 0)
    def _():
        st_ref[...] = s

    @pl.when(pl.program_id(0) != 0)
    def _():
        st_ref[...] = st_ref[...] + s


def _head_out_body(h_ref, st_ref, g_ref, bb_ref, wh2_ref, bh2_ref, out_ref, *, n):
    mu = st_ref[0:1, :] * (1.0 / n)
    var = st_ref[1:2, :] * (1.0 / n) - mu * mu
    h = (h_ref[...] - mu) / jnp.sqrt(var + 1e-5) * g_ref[...] + bb_ref[...]
    h = jnp.maximum(h, 0.0)
    out_ref[...] = _dot(h, wh2_ref[...]) + bh2_ref[...]


def _head_call(d0, wh1, bh1, g_bn, b_bn, wh2, bh2, bn):
    n, m = d0.shape
    mo = wh2.shape[1]
    grid = n // bn
    h, st = pl.pallas_call(
        _head_stats_body,
        grid=(grid,),
        in_specs=[
            pl.BlockSpec((bn, m), lambda i: (i, 0)),
            pl.BlockSpec((m, m), lambda i: (0, 0)),
            pl.BlockSpec((1, m), lambda i: (0, 0)),
        ],
        out_specs=[
            pl.BlockSpec((bn, m), lambda i: (i, 0)),
            pl.BlockSpec((8, m), lambda i: (0, 0)),
        ],
        out_shape=[
            jax.ShapeDtypeStruct((n, m), jnp.float32),
            jax.ShapeDtypeStruct((8, m), jnp.float32),
        ],
    )(d0, wh1, bh1.reshape(1, m))
    body = functools.partial(_head_out_body, n=n)
    return pl.pallas_call(
        body,
        grid=(grid,),
        in_specs=[
            pl.BlockSpec((bn, m), lambda i: (i, 0)),
            pl.BlockSpec((8, m), lambda i: (0, 0)),
            pl.BlockSpec((1, m), lambda i: (0, 0)),
            pl.BlockSpec((1, m), lambda i: (0, 0)),
            pl.BlockSpec((m, mo), lambda i: (0, 0)),
            pl.BlockSpec((1, mo), lambda i: (0, 0)),
        ],
        out_specs=pl.BlockSpec((bn, mo), lambda i: (i, 0)),
        out_shape=jax.ShapeDtypeStruct((n, mo), jnp.float32),
    )(h, st, g_bn.reshape(1, m), b_bn.reshape(1, m), wh2, bh2.reshape(1, mo))


def kernel(x, neigh0, neigh1, neigh2, sub0, sub1, up0, up1, obs,
           W0, b0, Wp0, bp0, W1, b1, Wp1, bp1, W2, b2,
           Wd1, bd1, Wd0, bd0, Wh1, bh1, g_bn, b_bn, Wh2, bh2):
    coords0 = x[:, :3]
    feats = x[:, 3:]
    obs8 = jnp.zeros((8, 16), jnp.float32).at[:3].set(obs)

    rows = _gather_table(_pack_table(feats, coords0), neigh0)
    f0 = _gib_call(rows, coords0, obs8, W0, b0, 1000)

    rows = _gather_table(_pack_table(f0, coords0), _pad_rows(sub0, _N1))
    coords1 = rows[0, :, 64:67]
    p1 = _gib_call(rows, coords1, obs8, Wp0, bp0, 640)

    rows = _gather_table(_pack_table(p1, coords1), _pad_rows(neigh1, _N1))
    f1 = _gib_call(rows, coords1, obs8, W1, b1, 640)

    rows = _gather_table(_pack_table(f1, coords1), _pad_rows(sub1, _N2))
    coords2 = rows[0, :, 128:131]
    p2 = _gib_call(rows, coords2, obs8, Wp1, bp1, 640)

    rows = _gather_table(_pack_table(p2, coords2), _pad_rows(neigh2, _N2))
    f2 = _gib_call(rows, coords2, obs8, W2, b2, 640)

    rows = _gather_table(_pack_table(f2, coords2), _pad_rows(up1, _N1))
    d1 = _unpool_call(rows, coords1, f1, Wd1, bd1, 320)

    rows = _gather_table(_pack_table(d1, coords1), up0)
    d0 = _unpool_call(rows, coords0, f0, Wd0, bd0, 400)

    return _head_call(d0, Wh1, bh1, g_bn, b_bn, Wh2, bh2, 400)


# speedup vs baseline: 1.2881x; 1.0284x over previous
import functools

import jax
import jax.numpy as jnp
from jax import lax
from jax.experimental import pallas as pl
from jax.experimental.pallas import tpu as pltpu
from jax.experimental.pallas import tpu_sc as plsc

_NC = 2
_NS = 16
_NW = _NC * _NS
_GRAN = 128 * _NW
_BUF_BYTES = 196608
_NEG_INV_2KS2 = -12.5

_N0, _N1, _N2 = 10000, 2560, 640


def _sc_gather(table, idx):
    _, D = table.shape
    B = idx.shape[0]
    bpw = B // _NW
    nch = 2
    while (bpw // nch) * D * 4 > _BUF_BYTES:
        nch *= 2
    chunk = bpw // nch
    idx3 = idx.reshape(_NW, nch, chunk)
    mesh = plsc.VectorSubcoreMesh(core_axis_name="c", subcore_axis_name="s")

    @functools.partial(
        pl.kernel,
        mesh=mesh,
        out_type=jax.ShapeDtypeStruct((B, D), jnp.float32),
        scratch_types=[
            pltpu.VMEM((nch, chunk), jnp.int32),
            pltpu.VMEM((chunk, D), jnp.float32),
            pltpu.VMEM((chunk, D), jnp.float32),
            pltpu.SemaphoreType.DMA,
            pltpu.SemaphoreType.DMA,
        ],
        compiler_params=pltpu.CompilerParams(use_tc_tiling_on_sc=False),
    )
    def gather_kernel(table_hbm, idx_hbm, out_hbm, idx_v, rows0, rows1, sem0, sem1):
        wid = lax.axis_index("s") * _NC + lax.axis_index("c")
        pltpu.sync_copy(idx_hbm.at[wid], idx_v)
        base = wid * bpw
        bufs = (rows0, rows1)
        sems = (sem0, sem1)
        copies = []
        for j in range(nch):
            copies.append(
                pltpu.async_copy(table_hbm.at[idx_v.at[j]], bufs[j % 2], sems[j % 2])
            )
            if j >= 1:
                copies[j - 1].wait()
                pltpu.sync_copy(
                    bufs[(j - 1) % 2],
                    out_hbm.at[pl.ds(base + (j - 1) * chunk, chunk)],
                )
        copies[nch - 1].wait()
        pltpu.sync_copy(
            bufs[(nch - 1) % 2],
            out_hbm.at[pl.ds(base + (nch - 1) * chunk, chunk)],
        )

    return gather_kernel(table, idx3)


def _gather_table(table, idx2d):
    n, k = idx2d.shape
    b = n * k
    idx = idx2d.T.reshape(-1).astype(jnp.int32)
    bp = -(-b // _GRAN) * _GRAN
    if bp > b:
        idx = jnp.concatenate([idx, jnp.zeros((bp - b,), jnp.int32)])
    rows = _sc_gather(table, idx)
    return rows[:b].reshape(k, n, table.shape[1])


def _pack_table(feats, coords):
    n = feats.shape[0]
    d = feats.shape[1] + 3
    dp = -(-d // 16) * 16
    return jnp.concatenate(
        [feats, coords, jnp.zeros((n, dp - d), jnp.float32)], axis=1
    )


def _pad_rows(a, n):
    return jnp.concatenate(
        [a, jnp.zeros((n - a.shape[0],) + a.shape[1:], a.dtype)]
    ) if a.shape[0] < n else a


def _bf(a):
    return a.astype(jnp.bfloat16).astype(jnp.float32)


def _dot(a, b):
    return jnp.dot(a.astype(jnp.bfloat16), b.astype(jnp.bfloat16),
                   preferred_element_type=jnp.float32)


def _gib_body(rows_ref, cdst_ref, obs_ref, w_ref, b_ref, out_ref, *, k, c, bn):
    cd = cdst_ref[...]
    o0 = _bf(obs_ref[0:1, :])
    o1 = _bf(obs_ref[1:2, :])
    o2 = _bf(obs_ref[2:3, :])

    def one(j):
        rj = rows_ref[pl.ds(j, 1)][0]
        rel = _bf(rj[:, c:c + 3] - cd)
        dj = rel[:, 0:1] * o0 + rel[:, 1:2] * o1 + rel[:, 2:3] * o2
        wj = _bf(jnp.exp(dj * dj * _NEG_INV_2KS2))
        fj = _bf(rj[:, 0:c])
        return wj, fj

    unroll = 8 if k % 8 == 0 else (4 if k % 4 == 0 else (2 if k % 2 == 0 else 1))

    def body(i, accs):
        wf = [one(unroll * i + u) for u in range(unroll)]
        for w, f in wf:
            accs = tuple(a + w[:, o:o + 1] * f for o, a in enumerate(accs))
        return accs

    steps = k // unroll

    init = tuple(jnp.zeros((bn, c), jnp.float32) for _ in range(16))
    accs = lax.fori_loop(0, steps, body, init)
    agg = jnp.concatenate(accs, axis=1) * (1.0 / k)
    out = _dot(agg, w_ref[...])
    out_ref[...] = jnp.maximum(out + b_ref[...], 0.0)


def _gib_call(rows3, cdst, obs8, w, b, bn):
    k, n, dp = rows3.shape
    c = w.shape[0] // 16
    m = w.shape[1]
    grid = n // bn
    body = functools.partial(_gib_body, k=k, c=c, bn=bn)
    return pl.pallas_call(
        body,
        grid=(grid,),
        in_specs=[
            pl.BlockSpec((k, bn, dp), lambda i: (0, i, 0)),
            pl.BlockSpec((bn, 3), lambda i: (i, 0)),
            pl.BlockSpec((8, 16), lambda i: (0, 0)),
            pl.BlockSpec((16 * c, m), lambda i: (0, 0)),
            pl.BlockSpec((1, m), lambda i: (0, 0)),
        ],
        out_specs=pl.BlockSpec((bn, m), lambda i: (i, 0)),
        out_shape=jax.ShapeDtypeStruct((n, m), jnp.float32),
    )(rows3, cdst, obs8, w, b.reshape(1, m))


def _unpool_body(rows_ref, clow_ref, skip_ref, wa_ref, wb_ref, b_ref, out_ref,
                 *, c):
    cl = clow_ref[...]
    ws = []
    for j in range(3):
        df = rows_ref[j][:, c:c + 3] - cl
        dd = jnp.sqrt(
            df[:, 0:1] * df[:, 0:1]
            + df[:, 1:2] * df[:, 1:2]
            + df[:, 2:3] * df[:, 2:3]
            + 1e-12
        )
        ws.append(1.0 / (dd + 1e-8))
    wsum = ws[0] + ws[1] + ws[2]
    interp = sum((ws[j] / wsum) * rows_ref[j][:, 0:c] for j in range(3))
    out = _dot(interp, wa_ref[...]) + _dot(skip_ref[...], wb_ref[...]) + b_ref[...]
    out_ref[...] = jnp.maximum(out, 0.0)


def _unpool_call(rows3, clow, skip, w, b, bn):
    _, n, dp = rows3.shape
    cs = skip.shape[1]
    c = w.shape[0] - cs
    m = w.shape[1]
    body = functools.partial(_unpool_body, c=c)
    return pl.pallas_call(
        body,
        grid=(n // bn,),
        in_specs=[
            pl.BlockSpec((3, bn, dp), lambda i: (0, i, 0)),
            pl.BlockSpec((bn, 3), lambda i: (i, 0)),
            pl.BlockSpec((bn, cs), lambda i: (i, 0)),
            pl.BlockSpec((c, m), lambda i: (0, 0)),
            pl.BlockSpec((cs, m), lambda i: (0, 0)),
            pl.BlockSpec((1, m), lambda i: (0, 0)),
        ],
        out_specs=pl.BlockSpec((bn, m), lambda i: (i, 0)),
        out_shape=jax.ShapeDtypeStruct((n, m), jnp.float32),
    )(rows3, clow, skip, w[:c], w[c:], b.reshape(1, m))


def _head_stats_body(d0_ref, wh1_ref, bh1_ref, h_ref, st_ref):
    h = _dot(d0_ref[...], wh1_ref[...]) + bh1_ref[...]
    h_ref[...] = h
    s = jnp.concatenate(
        [
            jnp.sum(h, axis=0, keepdims=True),
            jnp.sum(h * h, axis=0, keepdims=True),
            jnp.zeros((6, h.shape[1]), jnp.float32),
        ]
    )

    @pl.when(pl.program_id(0) == 0)
    def _():
        st_ref[...] = s

    @pl.when(pl.program_id(0) != 0)
    def _():
        st_ref[...] = st_ref[...] + s


def _head_out_body(h_ref, st_ref, g_ref, bb_ref, wh2_ref, bh2_ref, out_ref, *, n):
    mu = st_ref[0:1, :] * (1.0 / n)
    var = st_ref[1:2, :] * (1.0 / n) - mu * mu
    h = (h_ref[...] - mu) / jnp.sqrt(var + 1e-5) * g_ref[...] + bb_ref[...]
    h = jnp.maximum(h, 0.0)
    out_ref[...] = _dot(h, wh2_ref[...]) + bh2_ref[...]


def _head_call(d0, wh1, bh1, g_bn, b_bn, wh2, bh2, bn):
    n, m = d0.shape
    mo = wh2.shape[1]
    grid = n // bn
    h, st = pl.pallas_call(
        _head_stats_body,
        grid=(grid,),
        in_specs=[
            pl.BlockSpec((bn, m), lambda i: (i, 0)),
            pl.BlockSpec((m, m), lambda i: (0, 0)),
            pl.BlockSpec((1, m), lambda i: (0, 0)),
        ],
        out_specs=[
            pl.BlockSpec((bn, m), lambda i: (i, 0)),
            pl.BlockSpec((8, m), lambda i: (0, 0)),
        ],
        out_shape=[
            jax.ShapeDtypeStruct((n, m), jnp.float32),
            jax.ShapeDtypeStruct((8, m), jnp.float32),
        ],
    )(d0, wh1, bh1.reshape(1, m))
    body = functools.partial(_head_out_body, n=n)
    return pl.pallas_call(
        body,
        grid=(grid,),
        in_specs=[
            pl.BlockSpec((bn, m), lambda i: (i, 0)),
            pl.BlockSpec((8, m), lambda i: (0, 0)),
            pl.BlockSpec((1, m), lambda i: (0, 0)),
            pl.BlockSpec((1, m), lambda i: (0, 0)),
            pl.BlockSpec((m, mo), lambda i: (0, 0)),
            pl.BlockSpec((1, mo), lambda i: (0, 0)),
        ],
        out_specs=pl.BlockSpec((bn, mo), lambda i: (i, 0)),
        out_shape=jax.ShapeDtypeStruct((n, mo), jnp.float32),
    )(h, st, g_bn.reshape(1, m), b_bn.reshape(1, m), wh2, bh2.reshape(1, mo))


def kernel(x, neigh0, neigh1, neigh2, sub0, sub1, up0, up1, obs,
           W0, b0, Wp0, bp0, W1, b1, Wp1, bp1, W2, b2,
           Wd1, bd1, Wd0, bd0, Wh1, bh1, g_bn, b_bn, Wh2, bh2):
    coords0 = x[:, :3]
    feats = x[:, 3:]
    obs8 = jnp.zeros((8, 16), jnp.float32).at[:3].set(obs)

    rows = _gather_table(_pack_table(feats, coords0), neigh0)
    f0 = _gib_call(rows, coords0, obs8, W0, b0, 1000)

    rows = _gather_table(_pack_table(f0, coords0), _pad_rows(sub0, _N1))
    coords1 = rows[0, :, 64:67]
    p1 = _gib_call(rows, coords1, obs8, Wp0, bp0, 640)

    rows = _gather_table(_pack_table(p1, coords1), _pad_rows(neigh1, _N1))
    f1 = _gib_call(rows, coords1, obs8, W1, b1, 640)

    rows = _gather_table(_pack_table(f1, coords1), _pad_rows(sub1, _N2))
    coords2 = rows[0, :, 128:131]
    p2 = _gib_call(rows, coords2, obs8, Wp1, bp1, 640)

    rows = _gather_table(_pack_table(p2, coords2), _pad_rows(neigh2, _N2))
    f2 = _gib_call(rows, coords2, obs8, W2, b2, 640)

    rows = _gather_table(_pack_table(f2, coords2), _pad_rows(up1, _N1))
    d1 = _unpool_call(rows, coords1, f1, Wd1, bd1, 320)

    rows = _gather_table(_pack_table(d1, coords1), up0)
    d0 = _unpool_call(rows, coords0, f0, Wd0, bd0, 400)

    return _head_call(d0, Wh1, bh1, g_bn, b_bn, Wh2, bh2, 400)
